# baseline (speedup 1.0000x reference)
import functools

import jax
import jax.numpy as jnp
from jax import lax
from jax.experimental import pallas as pl
from jax.experimental.pallas import tpu as pltpu
from jax.experimental.pallas import tpu_sc as plsc

N = 10000
E = 320000
D = 128
H = 256
G = 64

C = 125
NCH = E // C
NSUB = 16
NCORE = 2
NP = 10240
STRIPE = NP // NSUB
IB = 40
BN = 1000
NBLK = N // BN

_MESH = plsc.VectorSubcoreMesh(core_axis_name="c", subcore_axis_name="s")
_PREC = lax.Precision.HIGHEST


def _deg128(col2d, ones_src, zrows):
    kpt = NCH // (NCORE * NSUB)

    @functools.partial(
        pl.kernel,
        out_type=jax.ShapeDtypeStruct((NCORE, NP, 128), jnp.float32),
        mesh=_MESH,
        scratch_types=[
            pltpu.VMEM((kpt, C), jnp.int32),
            pltpu.VMEM((C, 128), jnp.float32),
            pltpu.VMEM_SHARED((NP, 128), jnp.float32),
            pltpu.SemaphoreType.DMA,
        ],
    )
    def k(col_hbm, ones_hbm, z_hbm, out_hbm, colbuf, onesv, acc, sem):
        c = lax.axis_index("c")
        s = lax.axis_index("s")
        wid = c * NSUB + s
        pltpu.sync_copy(z_hbm, acc.at[pl.ds(s * STRIPE, STRIPE)])
        pltpu.sync_copy(ones_hbm, onesv)
        pltpu.sync_copy(col_hbm.at[pl.ds(wid * kpt, kpt)], colbuf)
        plsc.subcore_barrier()

        fk = 8

        @pl.loop(0, kpt // fk)
        def _(g):
            for u in range(fk):
                pltpu.async_copy(onesv, acc.at[colbuf.at[g * fk + u]], sem,
                                 add=True)
            for u in range(fk):
                pltpu.make_async_copy(onesv, acc.at[colbuf.at[g * fk + u]],
                                      sem).wait()

        plsc.subcore_barrier()
        pltpu.sync_copy(acc.at[pl.ds(s * STRIPE, STRIPE)],
                        out_hbm.at[c, pl.ds(s * STRIPE, STRIPE)])

    return k(col2d, ones_src, zrows)


def _db_pairs(y_hbm, rowbuf, colbuf, g0, g1, acc, sem0, sem1):
    def start(j, g, sem):
        pltpu.async_copy(y_hbm.at[rowbuf.at[j]], g, sem)

    def wait(j, g, sem):
        pltpu.make_async_copy(y_hbm.at[rowbuf.at[j]], g, sem).wait()

    start(0, g0, sem0)

    @pl.loop(0, IB // 2)
    def _(i):
        j0 = 2 * i
        wait(j0, g0, sem0)
        start(j0 + 1, g1, sem1)
        pltpu.sync_copy(g0, acc.at[colbuf.at[j0]], add=True)
        wait(j0 + 1, g1, sem1)

        @pl.when(i < IB // 2 - 1)
        def _():
            start(j0 + 2, g0, sem0)

        pltpu.sync_copy(g1, acc.at[colbuf.at[j0 + 1]], add=True)


def _propagate_edge_split(row2d, col2d, y, zrows):
    kpt = NCH // (NCORE * NSUB)
    nb = kpt // IB

    @functools.partial(
        pl.kernel,
        out_type=jax.ShapeDtypeStruct((NCORE, NP, 128), jnp.float32),
        mesh=_MESH,
        scratch_types=[
            pltpu.VMEM((IB, C), jnp.int32),
            pltpu.VMEM((IB, C), jnp.int32),
            pltpu.VMEM((C, 128), jnp.float32),
            pltpu.VMEM((C, 128), jnp.float32),
            pltpu.VMEM_SHARED((NP, 128), jnp.float32),
            pltpu.SemaphoreType.DMA,
            pltpu.SemaphoreType.DMA,
        ],
    )
    def k(row_hbm, col_hbm, y_hbm, z_hbm, out_hbm, rowbuf, colbuf,
          g0, g1, acc, sem0, sem1):
        c = lax.axis_index("c")
        s = lax.axis_index("s")
        wid = c * NSUB + s
        pltpu.sync_copy(z_hbm, acc.at[pl.ds(s * STRIPE, STRIPE)])
        plsc.subcore_barrier()

        @pl.loop(0, nb)
        def _(b):
            base = wid * kpt + b * IB
            pltpu.sync_copy(row_hbm.at[pl.ds(base, IB)], rowbuf)
            pltpu.sync_copy(col_hbm.at[pl.ds(base, IB)], colbuf)
            _db_pairs(y_hbm, rowbuf, colbuf, g0, g1, acc, sem0, sem1)

        plsc.subcore_barrier()
        pltpu.sync_copy(acc.at[pl.ds(s * STRIPE, STRIPE)],
                        out_hbm.at[c, pl.ds(s * STRIPE, STRIPE)])

    return k(row2d, col2d, y, zrows)


def _propagate_feat_split(rowpair, col2d, ycat, zrows):
    kps = NCH // NSUB
    nb = kps // IB

    @functools.partial(
        pl.kernel,
        out_type=jax.ShapeDtypeStruct((NCORE, NP, 128), jnp.float32),
        mesh=_MESH,
        scratch_types=[
            pltpu.VMEM((IB, C), jnp.int32),
            pltpu.VMEM((IB, C), jnp.int32),
            pltpu.VMEM((C, 128), jnp.float32),
            pltpu.VMEM((C, 128), jnp.float32),
            pltpu.VMEM_SHARED((NP, 128), jnp.float32),
            pltpu.SemaphoreType.DMA,
            pltpu.SemaphoreType.DMA,
        ],
    )
    def k(rowp_hbm, col_hbm, y_hbm, z_hbm, out_hbm, rowbuf, colbuf,
          g0, g1, acc, sem0, sem1):
        c = lax.axis_index("c")
        s = lax.axis_index("s")
        pltpu.sync_copy(z_hbm, acc.at[pl.ds(s * STRIPE, STRIPE)])
        plsc.subcore_barrier()

        @pl.loop(0, nb)
        def _(b):
            base = s * kps + b * IB
            pltpu.sync_copy(rowp_hbm.at[c, pl.ds(base, IB)], rowbuf)
            pltpu.sync_copy(col_hbm.at[pl.ds(base, IB)], colbuf)
            _db_pairs(y_hbm, rowbuf, colbuf, g0, g1, acc, sem0, sem1)

        plsc.subcore_barrier()
        pltpu.sync_copy(acc.at[pl.ds(s * STRIPE, STRIPE)],
                        out_hbm.at[c, pl.ds(s * STRIPE, STRIPE)])

    return k(rowpair, col2d, ycat, zrows)


def _dis_from_deg(deg_blk):
    d = deg_blk[0, :, 0] + deg_blk[1, :, 0] + 1.0
    r = lax.rsqrt(d)
    r = r * (1.5 - 0.5 * d * r * r)
    return r[:, None]


def _bf16r(a):
    return a.astype(jnp.bfloat16).astype(jnp.float32)


def _tc_scale(deg16, x):
    def body(deg_ref, x_ref, y_ref):
        dis = _dis_from_deg(deg_ref[...])
        y_ref[...] = _bf16r(x_ref[...]) * dis

    return pl.pallas_call(
        body,
        grid=(NBLK,),
        in_specs=[
            pl.BlockSpec((NCORE, BN, 128), lambda i: (0, i, 0)),
            pl.BlockSpec((BN, D), lambda i: (i, 0)),
        ],
        out_specs=pl.BlockSpec((BN, D), lambda i: (i, 0)),
        out_shape=jax.ShapeDtypeStruct((N, D), jnp.float32),
    )(deg16, x)


def _tc_layer1(S1, deg16, x, Wc1, bc1, Wc2):
    def body(S_ref, deg_ref, x_ref, w_ref, b_ref, w2_ref, ya_ref, yb_ref):
        dis = _dis_from_deg(deg_ref[...])
        s = S_ref[...]
        scat = s[0] + s[1]
        prop = dis * scat + (dis * dis) * _bf16r(x_ref[...])
        h = lax.dot_general(prop, _bf16r(w_ref[...]), (((1,), (0,)), ((), ())),
                            precision=_PREC,
                            preferred_element_type=jnp.float32)
        h = jnp.maximum(h + b_ref[...], 0.0)
        xw2 = lax.dot_general(_bf16r(h), _bf16r(w2_ref[...]),
                              (((1,), (0,)), ((), ())),
                              precision=_PREC,
                              preferred_element_type=jnp.float32)
        y2 = dis * xw2
        ya_ref[...] = y2[:, : H // 2]
        yb_ref[...] = y2[:, H // 2:]

    return pl.pallas_call(
        body,
        grid=(NBLK,),
        in_specs=[
            pl.BlockSpec((NCORE, BN, D), lambda i: (0, i, 0)),
            pl.BlockSpec((NCORE, BN, 128), lambda i: (0, i, 0)),
            pl.BlockSpec((BN, D), lambda i: (i, 0)),
            pl.BlockSpec((D, H), lambda i: (0, 0)),
            pl.BlockSpec((1, H), lambda i: (0, 0)),
            pl.BlockSpec((H, H), lambda i: (0, 0)),
        ],
        out_specs=[
            pl.BlockSpec((BN, H // 2), lambda i: (i, 0)),
            pl.BlockSpec((BN, H // 2), lambda i: (i, 0)),
        ],
        out_shape=[
            jax.ShapeDtypeStruct((N, H // 2), jnp.float32),
            jax.ShapeDtypeStruct((N, H // 2), jnp.float32),
        ],
    )(S1, deg16, x, Wc1, bc1.reshape(1, H), Wc2)


def _tc_pool_head(S2, y2a, y2b, deg16, bm3, bc2, Wl1, bl1, Wl2, bl2):
    def body(S_ref, ya_ref, yb_ref, deg_ref, bm_ref, b_ref,
             wl1_ref, bl1_ref, wl2_ref, bl2_ref, out_ref, pool_ref, cnt_ref):
        i = pl.program_id(0)

        @pl.when(i == 0)
        def _():
            pool_ref[...] = jnp.zeros_like(pool_ref)
            cnt_ref[...] = jnp.zeros_like(cnt_ref)

        dis = _dis_from_deg(deg_ref[...])
        s = S_ref[...]
        scat = jnp.concatenate([s[0], s[1]], axis=1)
        ycat = jnp.concatenate([ya_ref[...], yb_ref[...]], axis=1)
        h2 = jnp.maximum(dis * (scat + ycat) + b_ref[...], 0.0)

        bm = bm_ref[...][0, 0]
        gids = lax.broadcasted_iota(jnp.int32, (1, G), 1)
        onehot = (bm[:, None] == gids).astype(jnp.float32)
        pool_ref[...] += lax.dot_general(
            onehot, h2, (((0,), (0,)), ((), ())),
            precision=_PREC, preferred_element_type=jnp.float32)
        ones = jnp.ones((BN, 128), jnp.float32)
        cnt_ref[...] += lax.dot_general(
            onehot, ones, (((0,), (0,)), ((), ())),
            precision=_PREC, preferred_element_type=jnp.float32)

        @pl.when(i == NBLK - 1)
        def _():
            pool = pool_ref[...] / jnp.maximum(cnt_ref[...][:, 0:1], 1.0)
            p = lax.dot_general(_bf16r(pool), _bf16r(wl1_ref[...]),
                                (((1,), (1,)), ((), ())),
                                precision=_PREC,
                                preferred_element_type=jnp.float32)
            p = jnp.maximum(p + bl1_ref[...], 0.0)
            o = jnp.sum(_bf16r(p) * _bf16r(wl2_ref[...]), axis=1,
                        keepdims=True)
            out_ref[...] = o + bl2_ref[...]

    return pl.pallas_call(
        body,
        grid=(NBLK,),
        in_specs=[
            pl.BlockSpec((NCORE, BN, H // 2), lambda i: (0, i, 0)),
            pl.BlockSpec((BN, H // 2), lambda i: (i, 0)),
            pl.BlockSpec((BN, H // 2), lambda i: (i, 0)),
            pl.BlockSpec((NCORE, BN, 128), lambda i: (0, i, 0)),
            pl.BlockSpec((1, 1, BN), lambda i: (i, 0, 0)),
            pl.BlockSpec((1, H), lambda i: (0, 0)),
            pl.BlockSpec((128, H), lambda i: (0, 0)),
            pl.BlockSpec((1, 128), lambda i: (0, 0)),
            pl.BlockSpec((1, 128), lambda i: (0, 0)),
            pl.BlockSpec((1, 1), lambda i: (0, 0)),
        ],
        out_specs=pl.BlockSpec((G, 1), lambda i: (0, 0)),
        out_shape=jax.ShapeDtypeStruct((G, 1), jnp.float32),
        scratch_shapes=[
            pltpu.VMEM((G, H), jnp.float32),
            pltpu.VMEM((G, 128), jnp.float32),
        ],
    )(S2, y2a, y2b, deg16, bm3, bc2.reshape(1, H), Wl1,
      bl1.reshape(1, 128), Wl2, bl2.reshape(1, 1))


def kernel(x, edge_index, batch_map, Wc1, bc1, Wc2, bc2, Wl1, bl1, Wl2, bl2):
    row2d = edge_index[0].reshape(NCH, C)
    col2d = edge_index[1].reshape(NCH, C)
    bm3 = batch_map.reshape(NBLK, 1, BN)

    rowpair = jnp.stack([row2d, row2d + N])
    ones_src = jnp.ones((C, 128), jnp.float32)
    z128 = jnp.zeros((STRIPE, 128), jnp.float32)

    deg16 = _deg128(col2d, ones_src, z128)
    y1 = _tc_scale(deg16, x)
    S1 = _propagate_edge_split(row2d, col2d, y1, z128)
    y2a, y2b = _tc_layer1(S1, deg16, x, Wc1, bc1, Wc2)
    ycat = jnp.concatenate([y2a, y2b], axis=0)
    S2 = _propagate_feat_split(rowpair, col2d, ycat, z128)
    return _tc_pool_head(S2, y2a, y2b, deg16, bm3, bc2, Wl1, bl1, Wl2, bl2)


# speedup vs baseline: 1.0273x; 1.0273x over previous
import functools

import jax
import jax.numpy as jnp
from jax import lax
from jax.experimental import pallas as pl
from jax.experimental.pallas import tpu as pltpu
from jax.experimental.pallas import tpu_sc as plsc

N = 10000
E = 320000
D = 128
H = 256
G = 64

C = 125
NCH = E // C
NSUB = 16
NCORE = 2
NP = 10240
STRIPE = NP // NSUB
IB = 40
BN = 1000
NBLK = N // BN

_MESH = plsc.VectorSubcoreMesh(core_axis_name="c", subcore_axis_name="s")
_PREC = lax.Precision.HIGHEST


def _deg128(col2d, ones_src, zrows):
    kpt = NCH // (NCORE * NSUB)

    @functools.partial(
        pl.kernel,
        out_type=jax.ShapeDtypeStruct((NCORE, NP, 128), jnp.float32),
        mesh=_MESH,
        scratch_types=[
            pltpu.VMEM((kpt, C), jnp.int32),
            pltpu.VMEM((C, 128), jnp.float32),
            pltpu.VMEM_SHARED((NP, 128), jnp.float32),
            pltpu.SemaphoreType.DMA,
        ],
    )
    def k(col_hbm, ones_hbm, z_hbm, out_hbm, colbuf, onesv, acc, sem):
        c = lax.axis_index("c")
        s = lax.axis_index("s")
        wid = c * NSUB + s
        pltpu.sync_copy(z_hbm, acc.at[pl.ds(s * STRIPE, STRIPE)])
        pltpu.sync_copy(ones_hbm, onesv)
        pltpu.sync_copy(col_hbm.at[pl.ds(wid * kpt, kpt)], colbuf)
        plsc.subcore_barrier()

        fk = 8

        @pl.loop(0, kpt // fk)
        def _(g):
            for u in range(fk):
                pltpu.async_copy(onesv, acc.at[colbuf.at[g * fk + u]], sem,
                                 add=True)
            for u in range(fk):
                pltpu.make_async_copy(onesv, acc.at[colbuf.at[g * fk + u]],
                                      sem).wait()

        plsc.subcore_barrier()
        pltpu.sync_copy(acc.at[pl.ds(s * STRIPE, STRIPE)],
                        out_hbm.at[c, pl.ds(s * STRIPE, STRIPE)])

    return k(col2d, ones_src, zrows)


def _db_pairs(y_hbm, rowbuf, colbuf, g0, g1, acc, sem0, sem1):
    def start(j, g, sem):
        pltpu.async_copy(y_hbm.at[rowbuf.at[j]], g, sem)

    def wait(j, g, sem):
        pltpu.make_async_copy(y_hbm.at[rowbuf.at[j]], g, sem).wait()

    start(0, g0, sem0)

    @pl.loop(0, IB // 2)
    def _(i):
        j0 = 2 * i
        wait(j0, g0, sem0)
        start(j0 + 1, g1, sem1)
        pltpu.sync_copy(g0, acc.at[colbuf.at[j0]], add=True)
        wait(j0 + 1, g1, sem1)

        @pl.when(i < IB // 2 - 1)
        def _():
            start(j0 + 2, g0, sem0)

        pltpu.sync_copy(g1, acc.at[colbuf.at[j0 + 1]], add=True)


def _propagate_edge_split(row2d, col2d, y, zrows):
    kpt = NCH // (NCORE * NSUB)
    nb = kpt // IB

    @functools.partial(
        pl.kernel,
        out_type=jax.ShapeDtypeStruct((NCORE, NP, 128), jnp.float32),
        mesh=_MESH,
        scratch_types=[
            pltpu.VMEM((IB, C), jnp.int32),
            pltpu.VMEM((IB, C), jnp.int32),
            pltpu.VMEM((C, 128), jnp.float32),
            pltpu.VMEM((C, 128), jnp.float32),
            pltpu.VMEM_SHARED((NP, 128), jnp.float32),
            pltpu.SemaphoreType.DMA,
            pltpu.SemaphoreType.DMA,
        ],
    )
    def k(row_hbm, col_hbm, y_hbm, z_hbm, out_hbm, rowbuf, colbuf,
          g0, g1, acc, sem0, sem1):
        c = lax.axis_index("c")
        s = lax.axis_index("s")
        wid = c * NSUB + s
        pltpu.sync_copy(z_hbm, acc.at[pl.ds(s * STRIPE, STRIPE)])
        plsc.subcore_barrier()

        @pl.loop(0, nb)
        def _(b):
            base = wid * kpt + b * IB
            pltpu.sync_copy(row_hbm.at[pl.ds(base, IB)], rowbuf)
            pltpu.sync_copy(col_hbm.at[pl.ds(base, IB)], colbuf)
            _db_pairs(y_hbm, rowbuf, colbuf, g0, g1, acc, sem0, sem1)

        plsc.subcore_barrier()
        pltpu.sync_copy(acc.at[pl.ds(s * STRIPE, STRIPE)],
                        out_hbm.at[c, pl.ds(s * STRIPE, STRIPE)])

    return k(row2d, col2d, y, zrows)


def _propagate_feat_split(rowpair, col2d, ycat, zrows):
    kps = NCH // NSUB
    nb = kps // IB

    @functools.partial(
        pl.kernel,
        out_type=jax.ShapeDtypeStruct((NCORE, NP, 128), jnp.float32),
        mesh=_MESH,
        scratch_types=[
            pltpu.VMEM((IB, C), jnp.int32),
            pltpu.VMEM((IB, C), jnp.int32),
            pltpu.VMEM((C, 128), jnp.float32),
            pltpu.VMEM((C, 128), jnp.float32),
            pltpu.VMEM_SHARED((NP, 128), jnp.float32),
            pltpu.SemaphoreType.DMA,
            pltpu.SemaphoreType.DMA,
        ],
    )
    def k(rowp_hbm, col_hbm, y_hbm, z_hbm, out_hbm, rowbuf, colbuf,
          g0, g1, acc, sem0, sem1):
        c = lax.axis_index("c")
        s = lax.axis_index("s")
        pltpu.sync_copy(z_hbm, acc.at[pl.ds(s * STRIPE, STRIPE)])
        plsc.subcore_barrier()

        @pl.loop(0, nb)
        def _(b):
            base = s * kps + b * IB
            pltpu.sync_copy(rowp_hbm.at[c, pl.ds(base, IB)], rowbuf)
            pltpu.sync_copy(col_hbm.at[pl.ds(base, IB)], colbuf)
            _db_pairs(y_hbm, rowbuf, colbuf, g0, g1, acc, sem0, sem1)

        plsc.subcore_barrier()
        pltpu.sync_copy(acc.at[pl.ds(s * STRIPE, STRIPE)],
                        out_hbm.at[c, pl.ds(s * STRIPE, STRIPE)])

    return k(rowpair, col2d, ycat, zrows)


def _dis_from_deg(deg_blk):
    d = deg_blk[0, :, 0] + deg_blk[1, :, 0] + 1.0
    r = lax.rsqrt(d)
    r = r * (1.5 - 0.5 * d * r * r)
    return r[:, None]


def _bf16r(a):
    return a.astype(jnp.bfloat16).astype(jnp.float32)


def _tc_scale(deg16, x):
    def body(deg_ref, x_ref, y_ref):
        dis = _dis_from_deg(deg_ref[...])
        y_ref[...] = _bf16r(x_ref[...]) * dis

    return pl.pallas_call(
        body,
        grid=(NBLK,),
        in_specs=[
            pl.BlockSpec((NCORE, BN, 128), lambda i: (0, i, 0)),
            pl.BlockSpec((BN, D), lambda i: (i, 0)),
        ],
        out_specs=pl.BlockSpec((BN, D), lambda i: (i, 0)),
        out_shape=jax.ShapeDtypeStruct((N, D), jnp.float32),
    )(deg16, x)


def _tc_layer1(S1, deg16, x, Wc1, bc1, Wc2):
    def body(S_ref, deg_ref, x_ref, w_ref, b_ref, w2_ref, ya_ref, yb_ref):
        dis = _dis_from_deg(deg_ref[...])
        s = S_ref[...]
        scat = s[0] + s[1]
        prop = dis * scat + (dis * dis) * _bf16r(x_ref[...])
        h = lax.dot_general(prop, _bf16r(w_ref[...]), (((1,), (0,)), ((), ())),
                            precision=_PREC,
                            preferred_element_type=jnp.float32)
        h = jnp.maximum(h + b_ref[...], 0.0)
        xw2 = lax.dot_general(_bf16r(h), _bf16r(w2_ref[...]),
                              (((1,), (0,)), ((), ())),
                              precision=lax.Precision.DEFAULT,
                              preferred_element_type=jnp.float32)
        y2 = dis * xw2
        ya_ref[...] = y2[:, : H // 2]
        yb_ref[...] = y2[:, H // 2:]

    return pl.pallas_call(
        body,
        grid=(NBLK,),
        in_specs=[
            pl.BlockSpec((NCORE, BN, D), lambda i: (0, i, 0)),
            pl.BlockSpec((NCORE, BN, 128), lambda i: (0, i, 0)),
            pl.BlockSpec((BN, D), lambda i: (i, 0)),
            pl.BlockSpec((D, H), lambda i: (0, 0)),
            pl.BlockSpec((1, H), lambda i: (0, 0)),
            pl.BlockSpec((H, H), lambda i: (0, 0)),
        ],
        out_specs=[
            pl.BlockSpec((BN, H // 2), lambda i: (i, 0)),
            pl.BlockSpec((BN, H // 2), lambda i: (i, 0)),
        ],
        out_shape=[
            jax.ShapeDtypeStruct((N, H // 2), jnp.float32),
            jax.ShapeDtypeStruct((N, H // 2), jnp.float32),
        ],
    )(S1, deg16, x, Wc1, bc1.reshape(1, H), Wc2)


def _tc_pool_head(S2, y2a, y2b, deg16, bm3, bc2, Wl1, bl1, Wl2, bl2):
    def body(S_ref, ya_ref, yb_ref, deg_ref, bm_ref, b_ref,
             wl1_ref, bl1_ref, wl2_ref, bl2_ref, out_ref, pool_ref, cnt_ref):
        i = pl.program_id(0)

        @pl.when(i == 0)
        def _():
            pool_ref[...] = jnp.zeros_like(pool_ref)
            cnt_ref[...] = jnp.zeros_like(cnt_ref)

        dis = _dis_from_deg(deg_ref[...])
        s = S_ref[...]
        scat = jnp.concatenate([s[0], s[1]], axis=1)
        ycat = jnp.concatenate([ya_ref[...], yb_ref[...]], axis=1)
        h2 = jnp.maximum(dis * (scat + ycat) + b_ref[...], 0.0)

        bm = bm_ref[...][0, 0]
        gids = lax.broadcasted_iota(jnp.int32, (1, G), 1)
        onehot = (bm[:, None] == gids).astype(jnp.float32)
        pool_ref[...] += lax.dot_general(
            onehot, h2, (((0,), (0,)), ((), ())),
            precision=_PREC, preferred_element_type=jnp.float32)
        ones = jnp.ones((BN, 128), jnp.float32)
        cnt_ref[...] += lax.dot_general(
            onehot, ones, (((0,), (0,)), ((), ())),
            precision=_PREC, preferred_element_type=jnp.float32)

        @pl.when(i == NBLK - 1)
        def _():
            pool = pool_ref[...] / jnp.maximum(cnt_ref[...][:, 0:1], 1.0)
            p = lax.dot_general(_bf16r(pool), _bf16r(wl1_ref[...]),
                                (((1,), (1,)), ((), ())),
                                precision=lax.Precision.DEFAULT,
                                preferred_element_type=jnp.float32)
            p = jnp.maximum(p + bl1_ref[...], 0.0)
            o = jnp.sum(_bf16r(p) * _bf16r(wl2_ref[...]), axis=1,
                        keepdims=True)
            out_ref[...] = o + bl2_ref[...]

    return pl.pallas_call(
        body,
        grid=(NBLK,),
        in_specs=[
            pl.BlockSpec((NCORE, BN, H // 2), lambda i: (0, i, 0)),
            pl.BlockSpec((BN, H // 2), lambda i: (i, 0)),
            pl.BlockSpec((BN, H // 2), lambda i: (i, 0)),
            pl.BlockSpec((NCORE, BN, 128), lambda i: (0, i, 0)),
            pl.BlockSpec((1, 1, BN), lambda i: (i, 0, 0)),
            pl.BlockSpec((1, H), lambda i: (0, 0)),
            pl.BlockSpec((128, H), lambda i: (0, 0)),
            pl.BlockSpec((1, 128), lambda i: (0, 0)),
            pl.BlockSpec((1, 128), lambda i: (0, 0)),
            pl.BlockSpec((1, 1), lambda i: (0, 0)),
        ],
        out_specs=pl.BlockSpec((G, 1), lambda i: (0, 0)),
        out_shape=jax.ShapeDtypeStruct((G, 1), jnp.float32),
        scratch_shapes=[
            pltpu.VMEM((G, H), jnp.float32),
            pltpu.VMEM((G, 128), jnp.float32),
        ],
    )(S2, y2a, y2b, deg16, bm3, bc2.reshape(1, H), Wl1,
      bl1.reshape(1, 128), Wl2, bl2.reshape(1, 1))


def kernel(x, edge_index, batch_map, Wc1, bc1, Wc2, bc2, Wl1, bl1, Wl2, bl2):
    row2d = edge_index[0].reshape(NCH, C)
    col2d = edge_index[1].reshape(NCH, C)
    bm3 = batch_map.reshape(NBLK, 1, BN)

    rowpair = jnp.stack([row2d, row2d + N])
    ones_src = jnp.ones((C, 128), jnp.float32)
    z128 = jnp.zeros((STRIPE, 128), jnp.float32)

    deg16 = _deg128(col2d, ones_src, z128)
    y1 = _tc_scale(deg16, x)
    S1 = _propagate_edge_split(row2d, col2d, y1, z128)
    y2a, y2b = _tc_layer1(S1, deg16, x, Wc1, bc1, Wc2)
    ycat = jnp.concatenate([y2a, y2b], axis=0)
    S2 = _propagate_feat_split(rowpair, col2d, ycat, z128)
    return _tc_pool_head(S2, y2a, y2b, deg16, bm3, bc2, Wl1, bl1, Wl2, bl2)


# speedup vs baseline: 1.0327x; 1.0052x over previous
import functools

import jax
import jax.numpy as jnp
from jax import lax
from jax.experimental import pallas as pl
from jax.experimental.pallas import tpu as pltpu
from jax.experimental.pallas import tpu_sc as plsc

N = 10000
E = 320000
D = 128
H = 256
G = 64

C = 125
NCH = E // C
NSUB = 16
NCORE = 2
NP = 10240
STRIPE = NP // NSUB
IB = 40
BN = 1000
NBLK = N // BN

_MESH = plsc.VectorSubcoreMesh(core_axis_name="c", subcore_axis_name="s")
_PREC = lax.Precision.HIGHEST


def _deg128(col2d, ones_src, zrows):
    kpt = NCH // (NCORE * NSUB)

    @functools.partial(
        pl.kernel,
        out_type=jax.ShapeDtypeStruct((NCORE, NP, 128), jnp.float32),
        mesh=_MESH,
        scratch_types=[
            pltpu.VMEM((kpt, C), jnp.int32),
            pltpu.VMEM((C, 128), jnp.float32),
            pltpu.VMEM_SHARED((NP, 128), jnp.float32),
            pltpu.SemaphoreType.DMA,
        ],
    )
    def k(col_hbm, ones_hbm, z_hbm, out_hbm, colbuf, onesv, acc, sem):
        c = lax.axis_index("c")
        s = lax.axis_index("s")
        wid = c * NSUB + s
        pltpu.sync_copy(z_hbm, acc.at[pl.ds(s * STRIPE, STRIPE)])
        pltpu.sync_copy(ones_hbm, onesv)
        pltpu.sync_copy(col_hbm.at[pl.ds(wid * kpt, kpt)], colbuf)
        plsc.subcore_barrier()

        fk = 8

        @pl.loop(0, kpt // fk)
        def _(g):
            for u in range(fk):
                pltpu.async_copy(onesv, acc.at[colbuf.at[g * fk + u]], sem,
                                 add=True)
            for u in range(fk):
                pltpu.make_async_copy(onesv, acc.at[colbuf.at[g * fk + u]],
                                      sem).wait()

        plsc.subcore_barrier()
        pltpu.sync_copy(acc.at[pl.ds(s * STRIPE, STRIPE)],
                        out_hbm.at[c, pl.ds(s * STRIPE, STRIPE)])

    return k(col2d, ones_src, zrows)


def _db_pairs(y_hbm, rowbuf, colbuf, g0, g1, acc, sem0, sem1):
    def start(j, g, sem):
        pltpu.async_copy(y_hbm.at[rowbuf.at[j]], g, sem)

    def wait(j, g, sem):
        pltpu.make_async_copy(y_hbm.at[rowbuf.at[j]], g, sem).wait()

    start(0, g0, sem0)

    @pl.loop(0, IB // 2)
    def _(i):
        j0 = 2 * i
        wait(j0, g0, sem0)
        start(j0 + 1, g1, sem1)
        pltpu.sync_copy(g0, acc.at[colbuf.at[j0]], add=True)
        wait(j0 + 1, g1, sem1)

        @pl.when(i < IB // 2 - 1)
        def _():
            start(j0 + 2, g0, sem0)

        pltpu.sync_copy(g1, acc.at[colbuf.at[j0 + 1]], add=True)


def _propagate_edge_split(row2d, col2d, y, zrows):
    kpt = NCH // (NCORE * NSUB)
    nb = kpt // IB

    @functools.partial(
        pl.kernel,
        out_type=jax.ShapeDtypeStruct((NCORE, NP, 128), jnp.float32),
        mesh=_MESH,
        scratch_types=[
            pltpu.VMEM((IB, C), jnp.int32),
            pltpu.VMEM((IB, C), jnp.int32),
            pltpu.VMEM((C, 128), jnp.float32),
            pltpu.VMEM((C, 128), jnp.float32),
            pltpu.VMEM_SHARED((NP, 128), jnp.float32),
            pltpu.SemaphoreType.DMA,
            pltpu.SemaphoreType.DMA,
        ],
    )
    def k(row_hbm, col_hbm, y_hbm, z_hbm, out_hbm, rowbuf, colbuf,
          g0, g1, acc, sem0, sem1):
        c = lax.axis_index("c")
        s = lax.axis_index("s")
        wid = c * NSUB + s
        pltpu.sync_copy(z_hbm, acc.at[pl.ds(s * STRIPE, STRIPE)])
        plsc.subcore_barrier()

        @pl.loop(0, nb)
        def _(b):
            base = wid * kpt + b * IB
            pltpu.sync_copy(row_hbm.at[pl.ds(base, IB)], rowbuf)
            pltpu.sync_copy(col_hbm.at[pl.ds(base, IB)], colbuf)
            _db_pairs(y_hbm, rowbuf, colbuf, g0, g1, acc, sem0, sem1)

        plsc.subcore_barrier()
        pltpu.sync_copy(acc.at[pl.ds(s * STRIPE, STRIPE)],
                        out_hbm.at[c, pl.ds(s * STRIPE, STRIPE)])

    return k(row2d, col2d, y, zrows)


def _propagate_feat_split(rowpair, col2d, ycat, zrows):
    kps = NCH // NSUB
    nb = kps // IB

    @functools.partial(
        pl.kernel,
        out_type=jax.ShapeDtypeStruct((NCORE, NP, 128), jnp.float32),
        mesh=_MESH,
        scratch_types=[
            pltpu.VMEM((IB, C), jnp.int32),
            pltpu.VMEM((IB, C), jnp.int32),
            pltpu.VMEM((C, 128), jnp.float32),
            pltpu.VMEM((C, 128), jnp.float32),
            pltpu.VMEM_SHARED((NP, 128), jnp.float32),
            pltpu.SemaphoreType.DMA,
            pltpu.SemaphoreType.DMA,
        ],
    )
    def k(rowp_hbm, col_hbm, y_hbm, z_hbm, out_hbm, rowbuf, colbuf,
          g0, g1, acc, sem0, sem1):
        c = lax.axis_index("c")
        s = lax.axis_index("s")
        pltpu.sync_copy(z_hbm, acc.at[pl.ds(s * STRIPE, STRIPE)])
        plsc.subcore_barrier()

        @pl.loop(0, nb)
        def _(b):
            base = s * kps + b * IB
            pltpu.sync_copy(rowp_hbm.at[c, pl.ds(base, IB)], rowbuf)
            pltpu.sync_copy(col_hbm.at[pl.ds(base, IB)], colbuf)
            _db_pairs(y_hbm, rowbuf, colbuf, g0, g1, acc, sem0, sem1)

        plsc.subcore_barrier()
        pltpu.sync_copy(acc.at[pl.ds(s * STRIPE, STRIPE)],
                        out_hbm.at[c, pl.ds(s * STRIPE, STRIPE)])

    return k(rowpair, col2d, ycat, zrows)


def _dis_from_deg(deg_blk):
    d = deg_blk[0, :, 0] + deg_blk[1, :, 0] + 1.0
    r = lax.rsqrt(d)
    r = r * (1.5 - 0.5 * d * r * r)
    return r[:, None]


def _bf16r(a):
    return a.astype(jnp.bfloat16).astype(jnp.float32)


def _tc_scale(deg16, x):
    def body(deg_ref, x_ref, y_ref, dis_ref):
        dis = _dis_from_deg(deg_ref[...])
        y_ref[...] = _bf16r(x_ref[...]) * dis
        dis_ref[...] = dis

    return pl.pallas_call(
        body,
        grid=(NBLK,),
        in_specs=[
            pl.BlockSpec((NCORE, BN, 128), lambda i: (0, i, 0)),
            pl.BlockSpec((BN, D), lambda i: (i, 0)),
        ],
        out_specs=[
            pl.BlockSpec((BN, D), lambda i: (i, 0)),
            pl.BlockSpec((BN, 1), lambda i: (i, 0)),
        ],
        out_shape=[
            jax.ShapeDtypeStruct((N, D), jnp.float32),
            jax.ShapeDtypeStruct((N, 1), jnp.float32),
        ],
    )(deg16, x)


def _tc_layer1(S1, dis1, x, Wc1, bc1, Wc2):
    def body(S_ref, dis_ref, x_ref, w_ref, b_ref, w2_ref, ya_ref, yb_ref):
        dis = dis_ref[...]
        s = S_ref[...]
        scat = s[0] + s[1]
        prop = dis * scat + (dis * dis) * _bf16r(x_ref[...])
        h = lax.dot_general(prop, _bf16r(w_ref[...]), (((1,), (0,)), ((), ())),
                            precision=_PREC,
                            preferred_element_type=jnp.float32)
        h = jnp.maximum(h + b_ref[...], 0.0)
        xw2 = lax.dot_general(_bf16r(h), _bf16r(w2_ref[...]),
                              (((1,), (0,)), ((), ())),
                              precision=lax.Precision.DEFAULT,
                              preferred_element_type=jnp.float32)
        y2 = dis * xw2
        ya_ref[...] = y2[:, : H // 2]
        yb_ref[...] = y2[:, H // 2:]

    return pl.pallas_call(
        body,
        grid=(NBLK,),
        in_specs=[
            pl.BlockSpec((NCORE, BN, D), lambda i: (0, i, 0)),
            pl.BlockSpec((BN, 1), lambda i: (i, 0)),
            pl.BlockSpec((BN, D), lambda i: (i, 0)),
            pl.BlockSpec((D, H), lambda i: (0, 0)),
            pl.BlockSpec((1, H), lambda i: (0, 0)),
            pl.BlockSpec((H, H), lambda i: (0, 0)),
        ],
        out_specs=[
            pl.BlockSpec((BN, H // 2), lambda i: (i, 0)),
            pl.BlockSpec((BN, H // 2), lambda i: (i, 0)),
        ],
        out_shape=[
            jax.ShapeDtypeStruct((N, H // 2), jnp.float32),
            jax.ShapeDtypeStruct((N, H // 2), jnp.float32),
        ],
    )(S1, dis1, x, Wc1, bc1.reshape(1, H), Wc2)


def _tc_pool_head(S2, y2a, y2b, dis1, bm3, bc2, Wl1, bl1, Wl2, bl2):
    def body(S_ref, ya_ref, yb_ref, dis_ref, bm_ref, b_ref,
             wl1_ref, bl1_ref, wl2_ref, bl2_ref, out_ref, pool_ref, cnt_ref):
        i = pl.program_id(0)

        @pl.when(i == 0)
        def _():
            pool_ref[...] = jnp.zeros_like(pool_ref)
            cnt_ref[...] = jnp.zeros_like(cnt_ref)

        dis = dis_ref[...]
        s = S_ref[...]
        scat = jnp.concatenate([s[0], s[1]], axis=1)
        ycat = jnp.concatenate([ya_ref[...], yb_ref[...]], axis=1)
        h2 = jnp.maximum(dis * (scat + ycat) + b_ref[...], 0.0)

        bm = bm_ref[...][0, 0]
        gids = lax.broadcasted_iota(jnp.int32, (1, G), 1)
        onehot = (bm[:, None] == gids).astype(jnp.float32)
        pool_ref[...] += lax.dot_general(
            onehot, h2, (((0,), (0,)), ((), ())),
            precision=_PREC, preferred_element_type=jnp.float32)
        ones = jnp.ones((BN, 128), jnp.float32)
        cnt_ref[...] += lax.dot_general(
            onehot, ones, (((0,), (0,)), ((), ())),
            precision=_PREC, preferred_element_type=jnp.float32)

        @pl.when(i == NBLK - 1)
        def _():
            pool = pool_ref[...] / jnp.maximum(cnt_ref[...][:, 0:1], 1.0)
            p = lax.dot_general(_bf16r(pool), _bf16r(wl1_ref[...]),
                                (((1,), (1,)), ((), ())),
                                precision=lax.Precision.DEFAULT,
                                preferred_element_type=jnp.float32)
            p = jnp.maximum(p + bl1_ref[...], 0.0)
            o = jnp.sum(_bf16r(p) * _bf16r(wl2_ref[...]), axis=1,
                        keepdims=True)
            out_ref[...] = o + bl2_ref[...]

    return pl.pallas_call(
        body,
        grid=(NBLK,),
        in_specs=[
            pl.BlockSpec((NCORE, BN, H // 2), lambda i: (0, i, 0)),
            pl.BlockSpec((BN, H // 2), lambda i: (i, 0)),
            pl.BlockSpec((BN, H // 2), lambda i: (i, 0)),
            pl.BlockSpec((BN, 1), lambda i: (i, 0)),
            pl.BlockSpec((1, 1, BN), lambda i: (i, 0, 0)),
            pl.BlockSpec((1, H), lambda i: (0, 0)),
            pl.BlockSpec((128, H), lambda i: (0, 0)),
            pl.BlockSpec((1, 128), lambda i: (0, 0)),
            pl.BlockSpec((1, 128), lambda i: (0, 0)),
            pl.BlockSpec((1, 1), lambda i: (0, 0)),
        ],
        out_specs=pl.BlockSpec((G, 1), lambda i: (0, 0)),
        out_shape=jax.ShapeDtypeStruct((G, 1), jnp.float32),
        scratch_shapes=[
            pltpu.VMEM((G, H), jnp.float32),
            pltpu.VMEM((G, 128), jnp.float32),
        ],
    )(S2, y2a, y2b, dis1, bm3, bc2.reshape(1, H), Wl1,
      bl1.reshape(1, 128), Wl2, bl2.reshape(1, 1))


def kernel(x, edge_index, batch_map, Wc1, bc1, Wc2, bc2, Wl1, bl1, Wl2, bl2):
    row2d = edge_index[0].reshape(NCH, C)
    col2d = edge_index[1].reshape(NCH, C)
    bm3 = batch_map.reshape(NBLK, 1, BN)

    rowpair = jnp.stack([row2d, row2d + N])
    ones_src = jnp.ones((C, 128), jnp.float32)
    z128 = jnp.zeros((STRIPE, 128), jnp.float32)

    deg16 = _deg128(col2d, ones_src, z128)
    y1, dis1 = _tc_scale(deg16, x)
    S1 = _propagate_edge_split(row2d, col2d, y1, z128)
    y2a, y2b = _tc_layer1(S1, dis1, x, Wc1, bc1, Wc2)
    ycat = jnp.concatenate([y2a, y2b], axis=0)
    S2 = _propagate_feat_split(rowpair, col2d, ycat, z128)
    return _tc_pool_head(S2, y2a, y2b, dis1, bm3, bc2, Wl1, bl1, Wl2, bl2)


# speedup vs baseline: 1.0543x; 1.0209x over previous
import functools

import jax
import jax.numpy as jnp
from jax import lax
from jax.experimental import pallas as pl
from jax.experimental.pallas import tpu as pltpu
from jax.experimental.pallas import tpu_sc as plsc

N = 10000
E = 320000
D = 128
H = 256
G = 64

C = 125
NCH = E // C
NSUB = 16
NCORE = 2
NP = 10240
STRIPE = NP // NSUB
IB = 40
BN = 1000
NBLK = N // BN

_MESH = plsc.VectorSubcoreMesh(core_axis_name="c", subcore_axis_name="s")
_PREC = lax.Precision.HIGHEST


def _deg128(col2d, ones_src, zrows):
    kpt = NCH // (NCORE * NSUB)

    @functools.partial(
        pl.kernel,
        out_type=jax.ShapeDtypeStruct((NCORE, NP, 128), jnp.float32),
        mesh=_MESH,
        scratch_types=[
            pltpu.VMEM((kpt, C), jnp.int32),
            pltpu.VMEM((C, 128), jnp.float32),
            pltpu.VMEM_SHARED((NP, 128), jnp.float32),
            pltpu.SemaphoreType.DMA,
        ],
    )
    def k(col_hbm, ones_hbm, z_hbm, out_hbm, colbuf, onesv, acc, sem):
        c = lax.axis_index("c")
        s = lax.axis_index("s")
        wid = c * NSUB + s
        pltpu.sync_copy(z_hbm, acc.at[pl.ds(s * STRIPE, STRIPE)])
        pltpu.sync_copy(ones_hbm, onesv)
        pltpu.sync_copy(col_hbm.at[pl.ds(wid * kpt, kpt)], colbuf)
        plsc.subcore_barrier()

        fk = 8

        @pl.loop(0, kpt // fk)
        def _(g):
            for u in range(fk):
                pltpu.async_copy(onesv, acc.at[colbuf.at[g * fk + u]], sem,
                                 add=True)
            for u in range(fk):
                pltpu.make_async_copy(onesv, acc.at[colbuf.at[g * fk + u]],
                                      sem).wait()

        plsc.subcore_barrier()
        pltpu.sync_copy(acc.at[pl.ds(s * STRIPE, STRIPE)],
                        out_hbm.at[c, pl.ds(s * STRIPE, STRIPE)])

    return k(col2d, ones_src, zrows)


def _db_pairs(y_hbm, rowbuf, colbuf, g0, g1, acc, sem0, sem1):
    def start(j, g, sem):
        pltpu.async_copy(y_hbm.at[rowbuf.at[j]], g, sem)

    def wait(j, g, sem):
        pltpu.make_async_copy(y_hbm.at[rowbuf.at[j]], g, sem).wait()

    start(0, g0, sem0)

    @pl.loop(0, IB // 2)
    def _(i):
        j0 = 2 * i
        wait(j0, g0, sem0)
        start(j0 + 1, g1, sem1)
        pltpu.sync_copy(g0, acc.at[colbuf.at[j0]], add=True)
        wait(j0 + 1, g1, sem1)

        @pl.when(i < IB // 2 - 1)
        def _():
            start(j0 + 2, g0, sem0)

        pltpu.sync_copy(g1, acc.at[colbuf.at[j0 + 1]], add=True)


def _propagate_edge_split(row2d, col2d, y, zrows):
    kpt = NCH // (NCORE * NSUB)
    nb = kpt // IB

    @functools.partial(
        pl.kernel,
        out_type=jax.ShapeDtypeStruct((NCORE, NP, 128), jnp.float32),
        mesh=_MESH,
        scratch_types=[
            pltpu.VMEM((IB, C), jnp.int32),
            pltpu.VMEM((IB, C), jnp.int32),
            pltpu.VMEM((C, 128), jnp.float32),
            pltpu.VMEM((C, 128), jnp.float32),
            pltpu.VMEM_SHARED((NP, 128), jnp.float32),
            pltpu.SemaphoreType.DMA,
            pltpu.SemaphoreType.DMA,
        ],
    )
    def k(row_hbm, col_hbm, y_hbm, z_hbm, out_hbm, rowbuf, colbuf,
          g0, g1, acc, sem0, sem1):
        c = lax.axis_index("c")
        s = lax.axis_index("s")
        wid = c * NSUB + s
        pltpu.sync_copy(z_hbm, acc.at[pl.ds(s * STRIPE, STRIPE)])
        plsc.subcore_barrier()

        @pl.loop(0, nb)
        def _(b):
            base = wid * kpt + b * IB
            pltpu.sync_copy(row_hbm.at[pl.ds(base, IB)], rowbuf)
            pltpu.sync_copy(col_hbm.at[pl.ds(base, IB)], colbuf)
            _db_pairs(y_hbm, rowbuf, colbuf, g0, g1, acc, sem0, sem1)

        plsc.subcore_barrier()
        pltpu.sync_copy(acc.at[pl.ds(s * STRIPE, STRIPE)],
                        out_hbm.at[c, pl.ds(s * STRIPE, STRIPE)])

    return k(row2d, col2d, y, zrows)


def _propagate_feat_split(rowpair, col2d, ycat, zrows):
    kps = NCH // NSUB
    nb = kps // IB

    @functools.partial(
        pl.kernel,
        out_type=jax.ShapeDtypeStruct((NCORE, NP, 128), jnp.float32),
        mesh=_MESH,
        scratch_types=[
            pltpu.VMEM((IB, C), jnp.int32),
            pltpu.VMEM((IB, C), jnp.int32),
            pltpu.VMEM((C, 128), jnp.float32),
            pltpu.VMEM((C, 128), jnp.float32),
            pltpu.VMEM_SHARED((NP, 128), jnp.float32),
            pltpu.SemaphoreType.DMA,
            pltpu.SemaphoreType.DMA,
        ],
    )
    def k(rowp_hbm, col_hbm, y_hbm, z_hbm, out_hbm, rowbuf, colbuf,
          g0, g1, acc, sem0, sem1):
        c = lax.axis_index("c")
        s = lax.axis_index("s")
        pltpu.sync_copy(z_hbm, acc.at[pl.ds(s * STRIPE, STRIPE)])
        plsc.subcore_barrier()

        @pl.loop(0, nb)
        def _(b):
            base = s * kps + b * IB
            pltpu.sync_copy(rowp_hbm.at[c, pl.ds(base, IB)], rowbuf)
            pltpu.sync_copy(col_hbm.at[pl.ds(base, IB)], colbuf)
            _db_pairs(y_hbm, rowbuf, colbuf, g0, g1, acc, sem0, sem1)

        plsc.subcore_barrier()
        pltpu.sync_copy(acc.at[pl.ds(s * STRIPE, STRIPE)],
                        out_hbm.at[c, pl.ds(s * STRIPE, STRIPE)])

    return k(rowpair, col2d, ycat, zrows)


def _dis_from_deg(deg_blk):
    d = deg_blk[0, :, 0] + deg_blk[1, :, 0] + 1.0
    r = lax.rsqrt(d)
    r = r * (1.5 - 0.5 * d * r * r)
    return r[:, None]


def _bf16r(a):
    return a.astype(jnp.bfloat16).astype(jnp.float32)


def _tc_scale(deg16, x):
    def body(deg_ref, x_ref, y_ref, dis_ref):
        dis = _dis_from_deg(deg_ref[...])
        y_ref[...] = _bf16r(x_ref[...]) * dis
        dis_ref[...] = dis

    return pl.pallas_call(
        body,
        grid=(NBLK,),
        in_specs=[
            pl.BlockSpec((NCORE, BN, 128), lambda i: (0, i, 0)),
            pl.BlockSpec((BN, D), lambda i: (i, 0)),
        ],
        out_specs=[
            pl.BlockSpec((BN, D), lambda i: (i, 0)),
            pl.BlockSpec((BN, 1), lambda i: (i, 0)),
        ],
        out_shape=[
            jax.ShapeDtypeStruct((N, D), jnp.float32),
            jax.ShapeDtypeStruct((N, 1), jnp.float32),
        ],
    )(deg16, x)


def _tc_layer1(S1, dis1, x, Wc1, bc1, Wc2):
    def body(S_ref, dis_ref, x_ref, w_ref, b_ref, w2_ref, y_ref):
        dis = dis_ref[...]
        s = S_ref[...]
        scat = s[0] + s[1]
        prop = dis * scat + (dis * dis) * _bf16r(x_ref[...])
        h = lax.dot_general(prop, _bf16r(w_ref[...]), (((1,), (0,)), ((), ())),
                            precision=_PREC,
                            preferred_element_type=jnp.float32)
        h = jnp.maximum(h + b_ref[...], 0.0)
        xw2 = lax.dot_general(_bf16r(h), _bf16r(w2_ref[...]),
                              (((1,), (0,)), ((), ())),
                              precision=lax.Precision.DEFAULT,
                              preferred_element_type=jnp.float32)
        y2 = dis * xw2
        y_ref[0] = y2[:, : H // 2]
        y_ref[1] = y2[:, H // 2:]

    return pl.pallas_call(
        body,
        grid=(NBLK,),
        in_specs=[
            pl.BlockSpec((NCORE, BN, D), lambda i: (0, i, 0)),
            pl.BlockSpec((BN, 1), lambda i: (i, 0)),
            pl.BlockSpec((BN, D), lambda i: (i, 0)),
            pl.BlockSpec((D, H), lambda i: (0, 0)),
            pl.BlockSpec((1, H), lambda i: (0, 0)),
            pl.BlockSpec((H, H), lambda i: (0, 0)),
        ],
        out_specs=pl.BlockSpec((NCORE, BN, H // 2), lambda i: (0, i, 0)),
        out_shape=jax.ShapeDtypeStruct((NCORE, N, H // 2), jnp.float32),
    )(S1, dis1, x, Wc1, bc1.reshape(1, H), Wc2)


def _tc_pool_head(S2, y2s, dis1, bm3, bc2, Wl1, bl1, Wl2, bl2):
    def body(S_ref, y_ref, dis_ref, bm_ref, b_ref,
             wl1_ref, bl1_ref, wl2_ref, bl2_ref, out_ref, pool_ref, cnt_ref):
        i = pl.program_id(0)

        @pl.when(i == 0)
        def _():
            pool_ref[...] = jnp.zeros_like(pool_ref)
            cnt_ref[...] = jnp.zeros_like(cnt_ref)

        dis = dis_ref[...]
        s = S_ref[...]
        scat = jnp.concatenate([s[0], s[1]], axis=1)
        y = y_ref[...]
        ycat = jnp.concatenate([y[0], y[1]], axis=1)
        h2 = jnp.maximum(dis * (scat + ycat) + b_ref[...], 0.0)

        bm = bm_ref[...][0, 0]
        gids = lax.broadcasted_iota(jnp.int32, (1, G), 1)
        onehot = (bm[:, None] == gids).astype(jnp.float32)
        pool_ref[...] += lax.dot_general(
            onehot, h2, (((0,), (0,)), ((), ())),
            precision=_PREC, preferred_element_type=jnp.float32)
        ones = jnp.ones((BN, 128), jnp.float32)
        cnt_ref[...] += lax.dot_general(
            onehot, ones, (((0,), (0,)), ((), ())),
            precision=_PREC, preferred_element_type=jnp.float32)

        @pl.when(i == NBLK - 1)
        def _():
            pool = pool_ref[...] / jnp.maximum(cnt_ref[...][:, 0:1], 1.0)
            p = lax.dot_general(_bf16r(pool), _bf16r(wl1_ref[...]),
                                (((1,), (1,)), ((), ())),
                                precision=lax.Precision.DEFAULT,
                                preferred_element_type=jnp.float32)
            p = jnp.maximum(p + bl1_ref[...], 0.0)
            o = jnp.sum(_bf16r(p) * _bf16r(wl2_ref[...]), axis=1,
                        keepdims=True)
            out_ref[...] = o + bl2_ref[...]

    return pl.pallas_call(
        body,
        grid=(NBLK,),
        in_specs=[
            pl.BlockSpec((NCORE, BN, H // 2), lambda i: (0, i, 0)),
            pl.BlockSpec((NCORE, BN, H // 2), lambda i: (0, i, 0)),
            pl.BlockSpec((BN, 1), lambda i: (i, 0)),
            pl.BlockSpec((1, 1, BN), lambda i: (i, 0, 0)),
            pl.BlockSpec((1, H), lambda i: (0, 0)),
            pl.BlockSpec((128, H), lambda i: (0, 0)),
            pl.BlockSpec((1, 128), lambda i: (0, 0)),
            pl.BlockSpec((1, 128), lambda i: (0, 0)),
            pl.BlockSpec((1, 1), lambda i: (0, 0)),
        ],
        out_specs=pl.BlockSpec((G, 1), lambda i: (0, 0)),
        out_shape=jax.ShapeDtypeStruct((G, 1), jnp.float32),
        scratch_shapes=[
            pltpu.VMEM((G, H), jnp.float32),
            pltpu.VMEM((G, 128), jnp.float32),
        ],
    )(S2, y2s, dis1, bm3, bc2.reshape(1, H), Wl1,
      bl1.reshape(1, 128), Wl2, bl2.reshape(1, 1))


def kernel(x, edge_index, batch_map, Wc1, bc1, Wc2, bc2, Wl1, bl1, Wl2, bl2):
    row2d = edge_index[0].reshape(NCH, C)
    col2d = edge_index[1].reshape(NCH, C)
    bm3 = batch_map.reshape(NBLK, 1, BN)

    rowpair = jnp.stack([row2d, row2d + N])
    ones_src = jnp.ones((C, 128), jnp.float32)
    z128 = jnp.zeros((STRIPE, 128), jnp.float32)

    deg16 = _deg128(col2d, ones_src, z128)
    y1, dis1 = _tc_scale(deg16, x)
    S1 = _propagate_edge_split(row2d, col2d, y1, z128)
    y2s = _tc_layer1(S1, dis1, x, Wc1, bc1, Wc2)
    ycat = y2s.reshape(NCORE * N, H // 2)
    S2 = _propagate_feat_split(rowpair, col2d, ycat, z128)
    return _tc_pool_head(S2, y2s, dis1, bm3, bc2, Wl1, bl1, Wl2, bl2)
